# baseline (speedup 1.0000x reference)
import jax
import jax.numpy as jnp
from jax import lax
from jax.experimental import pallas as pl
from jax.experimental.pallas import tpu as pltpu
from jax.experimental.pallas import tpu_sc as plsc

NUM_FEATURES = 26
VOCAB = 100000
EMBED_DIM = 16
BATCH = 16384

NUM_WORKERS = 32
BC = BATCH // NUM_WORKERS
W = 128
NWIN = BC // W
SUPER_ROWS = NUM_FEATURES * VOCAB // 8


def kernel(categorical_features, tables):
    tab = tables.reshape(SUPER_ROWS, 128)
    cf_t = jnp.transpose(categorical_features.astype(jnp.int32))
    iota16 = jax.lax.iota(jnp.int32, 16)

    mesh = plsc.VectorSubcoreMesh(core_axis_name="c", subcore_axis_name="s")

    @pl.kernel(
        out_type=jax.ShapeDtypeStruct((NUM_FEATURES * EMBED_DIM, BATCH), jnp.float32),
        mesh=mesh,
        compiler_params=pltpu.CompilerParams(
            use_tc_tiling_on_sc=False, needs_layout_passes=False
        ),
        scratch_types=[
            pltpu.VMEM((BC,), jnp.int32),
            pltpu.VMEM((BC,), jnp.int32),
            pltpu.VMEM((BC,), jnp.int32),
            pltpu.VMEM((16,), jnp.int32),
            pltpu.VMEM((W, 128), jnp.float32),
            pltpu.VMEM((W, 128), jnp.float32),
            pltpu.VMEM((EMBED_DIM, BC), jnp.float32),
            pltpu.SemaphoreType.DMA,
            pltpu.SemaphoreType.DMA,
            pltpu.SemaphoreType.DMA,
        ],
    )
    def gather_kernel(
        tab_hbm, cf_hbm, iota_hbm, out_hbm,
        idx_v, g_v, r_v, iota_v, rows_a, rows_b, out_v, sem_a, sem_b, sem_o,
    ):
        wid = lax.axis_index("s") * 2 + lax.axis_index("c")
        b0 = wid * BC
        pltpu.sync_copy(iota_hbm, iota_v)
        jvec_base = iota_v[...]

        def extract(buf, wi):
            @pl.loop(0, W, step=16)
            def _(j2):
                jv = jvec_base + j2
                rv = r_v[pl.ds(wi * W + j2, 16)]
                for e in range(EMBED_DIM):
                    vals = plsc.load_gather(buf, [jv, rv + e])
                    out_v[e, pl.ds(wi * W + j2, 16)] = vals

        @pl.loop(0, NUM_FEATURES)
        def _(f):
            pltpu.sync_copy(cf_hbm.at[f, pl.ds(b0, BC)], idx_v)
            fbase = f * VOCAB

            @pl.loop(0, BC, step=16)
            def _(j):
                fr = idx_v[pl.ds(j, 16)] + fbase
                g_v[pl.ds(j, 16)] = jnp.right_shift(fr, 3)
                r_v[pl.ds(j, 16)] = jnp.left_shift(jnp.bitwise_and(fr, 7), 4)

            cp0 = pltpu.async_copy(tab_hbm.at[g_v.at[pl.ds(0 * W, W)]], rows_a, sem_a)
            cp1 = pltpu.async_copy(tab_hbm.at[g_v.at[pl.ds(1 * W, W)]], rows_b, sem_b)

            @pl.when(f != 0)
            def _():
                pltpu.make_async_copy(
                    out_v, out_hbm.at[pl.ds(0, EMBED_DIM), pl.ds(b0, BC)], sem_o
                ).wait()

            cp0.wait()
            extract(rows_a, 0)
            cp2 = pltpu.async_copy(tab_hbm.at[g_v.at[pl.ds(2 * W, W)]], rows_a, sem_a)
            cp1.wait()
            extract(rows_b, 1)
            cp3 = pltpu.async_copy(tab_hbm.at[g_v.at[pl.ds(3 * W, W)]], rows_b, sem_b)
            cp2.wait()
            extract(rows_a, 2)
            cp3.wait()
            extract(rows_b, 3)

            pltpu.async_copy(
                out_v,
                out_hbm.at[pl.ds(f * EMBED_DIM, EMBED_DIM), pl.ds(b0, BC)],
                sem_o,
            )

        pltpu.make_async_copy(
            out_v,
            out_hbm.at[pl.ds((NUM_FEATURES - 1) * EMBED_DIM, EMBED_DIM), pl.ds(b0, BC)],
            sem_o,
        ).wait()

    out_t = gather_kernel(tab, cf_t, iota16)
    return jnp.transpose(out_t)


# speedup vs baseline: 1.2479x; 1.2479x over previous
import jax
import jax.numpy as jnp
from jax.experimental import pallas as pl
from jax.experimental.pallas import tpu as pltpu
from jax.experimental.pallas import tpu_sc as plsc

NUM_FEATURES = 26
VOCAB = 100000
EMBED_DIM = 16
BATCH = 16384
NUM_IDX = BATCH * NUM_FEATURES
WINDOW = 128
SUPER_ROWS = NUM_FEATURES * VOCAB // 8
ROWS_PER_F = VOCAB // 8
CHUNK_V = 4000
CHUNKS = VOCAB // CHUNK_V


def _repack_tables(tables):
    tab_t = jnp.transpose(tables, (0, 2, 1)).reshape(
        NUM_FEATURES // 2, 2, EMBED_DIM, VOCAB
    )

    def body(in_ref, out_ref):
        for ff in range(2):
            for j in range(8):
                x = in_ref[0, ff, :, pl.ds(j * ROWS_PER_F, ROWS_PER_F)]
                out_ref[
                    0,
                    pl.ds(ff * ROWS_PER_F, ROWS_PER_F),
                    pl.ds(j * EMBED_DIM, EMBED_DIM),
                ] = jnp.transpose(x)

    out = pl.pallas_call(
        body,
        grid=(NUM_FEATURES // 2,),
        in_specs=[
            pl.BlockSpec((1, 2, EMBED_DIM, VOCAB), lambda f: (f, 0, 0, 0))
        ],
        out_specs=pl.BlockSpec((1, 2 * ROWS_PER_F, 128), lambda f: (f, 0, 0)),
        out_shape=jax.ShapeDtypeStruct(
            (NUM_FEATURES // 2, 2 * ROWS_PER_F, 128), jnp.float32
        ),
        compiler_params=pltpu.CompilerParams(
            dimension_semantics=("parallel",)
        ),
    )(tab_t)
    return out.reshape(SUPER_ROWS, 128)


def kernel(categorical_features, tables):
    flat_tables = _repack_tables(tables).reshape(NUM_FEATURES * VOCAB, EMBED_DIM)
    offs = jnp.arange(NUM_FEATURES, dtype=jnp.int32) * ROWS_PER_F
    v = categorical_features.astype(jnp.int32)
    j, vm = jnp.divmod(v, ROWS_PER_F)
    flat_idx = (8 * (offs[None, :] + vm) + j).reshape(1, NUM_IDX)

    mesh = plsc.VectorSubcoreMesh(core_axis_name="core", subcore_axis_name="subcore")

    @pl.kernel(
        out_type=jax.ShapeDtypeStruct((NUM_IDX, EMBED_DIM), jnp.float32),
        mesh=mesh,
        compiler_params=pltpu.CompilerParams(use_tc_tiling_on_sc=False),
    )
    def gather_kernel(table_hbm, idx_hbm, out_hbm):
        def body(idx_vmem, out_vmem):
            pltpu.sync_copy(table_hbm.at[idx_vmem.at[0]], out_vmem)

        pltpu.emit_pipeline(
            body,
            grid=(NUM_IDX // WINDOW,),
            in_specs=[pl.BlockSpec((1, WINDOW), index_map=lambda i: (0, i))],
            out_specs=[pl.BlockSpec((WINDOW, EMBED_DIM), index_map=lambda i: (i, 0))],
            core_axis_name=("core", "subcore"),
            dimension_semantics=(pltpu.PARALLEL,),
        )(idx_hbm, out_hbm)

    out = gather_kernel(flat_tables, flat_idx)
    return out.reshape(BATCH, NUM_FEATURES * EMBED_DIM)


# speedup vs baseline: 4.1014x; 3.2865x over previous
import jax
import jax.numpy as jnp
from jax.experimental import pallas as pl
from jax.experimental.pallas import tpu as pltpu
from jax.experimental.pallas import tpu_sc as plsc

NUM_FEATURES = 26
VOCAB = 100000
EMBED_DIM = 16
BATCH = 16384
NUM_IDX = BATCH * NUM_FEATURES
WINDOW = 128
SUPER_ROWS = NUM_FEATURES * VOCAB // 8
ROWS_PER_F = VOCAB // 8
CHUNK_V = 4000
CHUNKS = VOCAB // CHUNK_V


def _repack_tables(tables):
    tab_t = jnp.transpose(tables, (0, 2, 1)).reshape(
        NUM_FEATURES // 2, 2, EMBED_DIM, VOCAB
    )

    def body(in_ref, out_ref):
        vcc = ROWS_PER_F // 4
        for ff in range(2):
            for c in range(4):
                t = jnp.concatenate(
                    [
                        in_ref[0, ff, :, pl.ds(j * ROWS_PER_F + c * vcc, vcc)]
                        for j in range(8)
                    ],
                    axis=0,
                )
                out_ref[
                    0, pl.ds(ff * ROWS_PER_F + c * vcc, vcc), :
                ] = jnp.transpose(t)

    out = pl.pallas_call(
        body,
        grid=(NUM_FEATURES // 2,),
        in_specs=[
            pl.BlockSpec((1, 2, EMBED_DIM, VOCAB), lambda f: (f, 0, 0, 0))
        ],
        out_specs=pl.BlockSpec((1, 2 * ROWS_PER_F, 128), lambda f: (f, 0, 0)),
        out_shape=jax.ShapeDtypeStruct(
            (NUM_FEATURES // 2, 2 * ROWS_PER_F, 128), jnp.float32
        ),
        compiler_params=pltpu.CompilerParams(
            dimension_semantics=("parallel",),
            vmem_limit_bytes=64 * 1024 * 1024,
        ),
    )(tab_t)
    return out.reshape(SUPER_ROWS, 128)


def kernel(categorical_features, tables):
    flat_tables = _repack_tables(tables).reshape(NUM_FEATURES * VOCAB, EMBED_DIM)
    offs = jnp.arange(NUM_FEATURES, dtype=jnp.int32) * ROWS_PER_F
    v = categorical_features.astype(jnp.int32)
    j, vm = jnp.divmod(v, ROWS_PER_F)
    flat_idx = (8 * (offs[None, :] + vm) + j).reshape(1, NUM_IDX)

    mesh = plsc.VectorSubcoreMesh(core_axis_name="core", subcore_axis_name="subcore")

    @pl.kernel(
        out_type=jax.ShapeDtypeStruct((NUM_IDX, EMBED_DIM), jnp.float32),
        mesh=mesh,
        compiler_params=pltpu.CompilerParams(use_tc_tiling_on_sc=False),
    )
    def gather_kernel(table_hbm, idx_hbm, out_hbm):
        def body(idx_vmem, out_vmem):
            pltpu.sync_copy(table_hbm.at[idx_vmem.at[0]], out_vmem)

        pltpu.emit_pipeline(
            body,
            grid=(NUM_IDX // WINDOW,),
            in_specs=[pl.BlockSpec((1, WINDOW), index_map=lambda i: (0, i))],
            out_specs=[pl.BlockSpec((WINDOW, EMBED_DIM), index_map=lambda i: (i, 0))],
            core_axis_name=("core", "subcore"),
            dimension_semantics=(pltpu.PARALLEL,),
        )(idx_hbm, out_hbm)

    out = gather_kernel(flat_tables, flat_idx)
    return out.reshape(BATCH, NUM_FEATURES * EMBED_DIM)
